# baseline (speedup 1.0000x reference)
import functools

import jax
import jax.numpy as jnp
from jax import lax
from jax.experimental import pallas as pl
from jax.experimental.pallas import tpu as pltpu
from jax.experimental.pallas import tpu_sc as plsc

NC = 2
NS = 16
NW = NC * NS
LANES = 16
CHUNK = 128
ROUND = 4


def _make_lookup(n_rows, n_tab, d, dtype):
    per_w = n_rows // NW
    n_round = per_w // (CHUNK * ROUND)
    mesh = plsc.VectorSubcoreMesh(core_axis_name="c", subcore_axis_name="s")

    @functools.partial(
        pl.kernel,
        out_type=jax.ShapeDtypeStruct((n_rows, d), dtype),
        mesh=mesh,
        scratch_types=[
            pltpu.VMEM((n_tab, d), dtype),
            pltpu.VMEM((per_w,), jnp.int32),
            pltpu.VMEM((CHUNK, d), dtype),
            pltpu.VMEM((CHUNK, d), dtype),
            pltpu.VMEM((CHUNK, d), dtype),
            pltpu.SemaphoreType.DMA,
            pltpu.SemaphoreType.DMA,
            pltpu.SemaphoreType.DMA,
            pltpu.SemaphoreType.DMA,
        ],
    )
    def lookup(z_hbm, tab_hbm, out_hbm, tab_v, idx_v,
               cb0, cb1, db, cp0, cp1, dg, dp):
        wid = lax.axis_index("s") * NC + lax.axis_index("c")
        base = wid * per_w

        pltpu.sync_copy(tab_hbm, tab_v)
        pltpu.sync_copy(z_hbm.at[pl.ds(base, per_w)], idx_v)

        @plsc.parallel_loop(0, per_w, step=LANES)
        def fix(i):
            sl = pl.ds(i, LANES)
            v = idx_v[sl]
            idx_v[sl] = jnp.where(v > -1, v + 1, n_tab - 1)

        def do_chunk(j, buf):
            cb = j * CHUNK

            @plsc.parallel_loop(0, CHUNK, step=LANES)
            def group(gb):
                zvec = idx_v[pl.ds(cb + gb, LANES)]
                for l in range(LANES):
                    row = zvec[l]
                    for jj in range(d // LANES):
                        buf[gb + l, pl.ds(jj * LANES, LANES)] = (
                            tab_v[row, pl.ds(jj * LANES, LANES)])

        def put(j, buf, sem):
            pltpu.async_copy(
                buf, out_hbm.at[pl.ds(base + j * CHUNK, CHUNK)], sem)

        def wait_put(buf, sem):
            pltpu.make_async_copy(
                buf, out_hbm.at[pl.ds(base, CHUNK)], sem).wait()

        def body(cc, carry):
            j0 = cc * ROUND

            for t in range(ROUND):
                buf, sem = (cb0, cp0) if t % 2 == 0 else (cb1, cp1)
                if t < 2:
                    @pl.when(cc > 0)
                    def _():
                        wait_put(buf, sem)
                else:
                    wait_put(buf, sem)
                do_chunk(j0 + t, buf)
                put(j0 + t, buf, sem)

            return carry

        lax.fori_loop(0, n_round, body, 0)
        for buf, sem in ((cb0, cp0), (cb1, cp1)):
            wait_put(buf, sem)

    return lookup


def kernel(z, r, table):
    b, a = z.shape
    n_tab, d = table.shape
    zf = z.reshape(-1).astype(jnp.int32)
    tpad = jnp.concatenate([table, jnp.zeros((1, d), table.dtype)], axis=0)
    emb = _make_lookup(b * a, n_tab + 1, d, table.dtype)(zf, tpad)
    return (z, r, emb.reshape(b, a, d))


# speedup vs baseline: 1.1707x; 1.1707x over previous
import functools

import jax
import jax.numpy as jnp
from jax import lax
from jax.experimental import pallas as pl
from jax.experimental.pallas import tpu as pltpu
from jax.experimental.pallas import tpu_sc as plsc

NC = 2
NS = 16
NW = NC * NS
LANES = 16
CHUNK = 128


def _make_lookup(n_rows, n_tab, d, dtype):
    per_w = n_rows // NW
    n_chunk = per_w // CHUNK
    tab_words = n_tab * d
    groups = CHUNK // LANES
    mesh = plsc.VectorSubcoreMesh(core_axis_name="c", subcore_axis_name="s")

    @functools.partial(
        pl.kernel,
        out_type=jax.ShapeDtypeStruct((n_rows * d,), dtype),
        mesh=mesh,
        scratch_types=[
            pltpu.VMEM((tab_words,), dtype),
            pltpu.VMEM((per_w,), jnp.int32),
            pltpu.VMEM((CHUNK * d,), dtype),
            pltpu.VMEM((CHUNK * d,), dtype),
            pltpu.SemaphoreType.DMA,
            pltpu.SemaphoreType.DMA,
        ],
    )
    def lookup(z_hbm, tabf_hbm, out_hbm, tab_v, idx_v, rows0, rows1, p0, p1):
        wid = lax.axis_index("s") * NC + lax.axis_index("c")
        base = wid * per_w

        pltpu.sync_copy(tabf_hbm, tab_v)
        pltpu.sync_copy(z_hbm.at[pl.ds(base, per_w)], idx_v)

        @plsc.parallel_loop(0, per_w, step=LANES)
        def fix(i):
            sl = pl.ds(i, LANES)
            v = idx_v[sl]
            idx_v[sl] = jnp.where(v > -1, (v + 1) * d, (n_tab - 1) * d)

        def do_chunk(j, buf):
            cb = j * CHUNK

            @plsc.parallel_loop(0, CHUNK, step=LANES)
            def group(gb):
                zvec = idx_v[pl.ds(cb + gb, LANES)]
                gbd = gb * d
                for l in range(LANES):
                    off = zvec[l]
                    o = gbd + l * d
                    for jj in range(d // LANES):
                        buf[pl.ds(o + jj * LANES, LANES)] = (
                            tab_v[pl.ds(off + jj * LANES, LANES)])

        def put(j, buf, sem):
            pltpu.async_copy(
                buf,
                out_hbm.at[pl.ds((base + j * CHUNK) * d, CHUNK * d)],
                sem)

        def wait_put(buf, sem):
            pltpu.make_async_copy(
                buf, out_hbm.at[pl.ds(base * d, CHUNK * d)], sem).wait()

        def body(cc, carry):
            for b, (buf, sem) in enumerate(((rows0, p0), (rows1, p1))):
                @pl.when(cc > 0)
                def _():
                    wait_put(buf, sem)

                do_chunk(2 * cc + b, buf)
                put(2 * cc + b, buf, sem)
            return carry

        lax.fori_loop(0, n_chunk // 2, body, 0)
        wait_put(rows0, p0)
        wait_put(rows1, p1)

    return lookup


def kernel(z, r, table):
    b, a = z.shape
    n_tab, d = table.shape
    zf = z.reshape(-1).astype(jnp.int32)
    tpad = jnp.concatenate([table, jnp.zeros((1, d), table.dtype)], axis=0)
    emb = _make_lookup(b * a, n_tab + 1, d, table.dtype)(zf, tpad.reshape(-1))
    return (z, r, emb.reshape(b, a, d))
